# baseline (speedup 1.0000x reference)
import functools

import jax
import jax.numpy as jnp
import numpy as np
from jax import lax
from jax.experimental import pallas as pl
from jax.experimental.pallas import tpu as pltpu
from jax.experimental.pallas import tpu_sc as plsc

VOCAB = 1000000
D = 64
HID = 128
NCLS = 2
B = 4096
L = 200

NW = 32
SPW = B // NW
WINDOW = 16

_mesh = plsc.VectorSubcoreMesh(core_axis_name="c", subcore_axis_name="s")


@functools.partial(
    pl.kernel,
    out_type=jax.ShapeDtypeStruct((B, D), jnp.float32),
    mesh=_mesh,
    scratch_types=[
        pltpu.VMEM((SPW, L), jnp.int32),
        pltpu.VMEM((L, SPW), jnp.int32),
        pltpu.VMEM((SPW, D), jnp.float32),
        pltpu.SemaphoreType.DMA,
    ],
    compiler_params=pltpu.CompilerParams(use_tc_tiling_on_sc=False,
                                         needs_layout_passes=False),
)
def _sc_pool(x_hbm, emb_hbm, out_hbm, xb_v, idx_v, acc_v, sem):
    wid = lax.axis_index("s") * 2 + lax.axis_index("c")
    pltpu.sync_copy(x_hbm.at[pl.ds(wid * SPW, SPW)], xb_v)

    lane = lax.iota(jnp.int32, 16)

    @pl.loop(0, L)
    def _tr(r):
        col = jnp.full((16,), r, jnp.int32)
        for gi in range(SPW // 16):
            rows = lane + (gi * 16)
            v = plsc.load_gather(xb_v, [rows, col])
            idx_v[r, pl.ds(gi * 16, 16)] = v

    pltpu.sync_copy(emb_hbm.at[idx_v.at[0]], acc_v)

    @pl.loop(0, L - 1)
    def _fire(i):
        pltpu.async_copy(emb_hbm.at[idx_v.at[i + 1]], acc_v, sem, add=True)

        @pl.when(i >= WINDOW - 1)
        def _():
            pltpu.make_async_copy(emb_hbm.at[idx_v.at[0]], acc_v, sem).wait()

    @pl.loop(0, WINDOW - 1)
    def _drain(_):
        pltpu.make_async_copy(emb_hbm.at[idx_v.at[0]], acc_v, sem).wait()

    pltpu.sync_copy(acc_v, out_hbm.at[pl.ds(wid * SPW, SPW)])


def _mlp_body(s_ref, w1_ref, b1_ref, w2_ref, b2_ref, o_ref):
    x = s_ref[...] * np.float32(1.0 / L)
    h = lax.dot_general(x, w1_ref[...], (((1,), (1,)), ((), ())),
                        preferred_element_type=jnp.float32)
    h = jnp.maximum(h + b1_ref[...], 0.0)
    o_ref[...] = lax.dot_general(h, w2_ref[...], (((1,), (1,)), ((), ())),
                                 preferred_element_type=jnp.float32) + b2_ref[...]


def _mlp(sums, W1, b1, W2, b2):
    return pl.pallas_call(
        _mlp_body,
        out_shape=jax.ShapeDtypeStruct((B, NCLS), jnp.float32),
    )(sums, W1, b1.reshape(1, HID), W2, b2.reshape(1, NCLS))


def kernel(x_in, emb, W1, b1, W2, b2):
    sums = _sc_pool(x_in, emb)
    return _mlp(sums, W1, b1, W2, b2)
